# baseline (speedup 1.0000x reference)
import jax
import jax.numpy as jnp
from jax.experimental import pallas as pl

_BT = 128


def _body(w_ref, xt_ref, o_ref):
    o_ref[...] = jax.lax.dot_general(
        w_ref[...], xt_ref[...],
        dimension_numbers=(((1,), (0,)), ((), ())),
        preferred_element_type=jnp.float32,
    )


def kernel(x, weight):
    B, D = x.shape
    C = weight.shape[0]
    xt = x.T
    out = pl.pallas_call(
        _body,
        grid=(B // _BT,),
        in_specs=[
            pl.BlockSpec((C, D), lambda j: (0, 0)),
            pl.BlockSpec((D, _BT), lambda j: (0, j)),
        ],
        out_specs=pl.BlockSpec((C, _BT), lambda j: (0, j)),
        out_shape=jax.ShapeDtypeStruct((C, B), jnp.float32),
    )(weight, xt)
    return out.T
